# Initial kernel scaffold; baseline (speedup 1.0000x reference)
import jax
import jax.numpy as jnp
from jax.experimental import pallas as pl


def kernel(query, key_t, value):
    raise NotImplementedError("write your pallas kernel here")


# speedup vs baseline: 2.1243x; 2.1243x over previous
import jax
import jax.numpy as jnp
from jax.experimental import pallas as pl

_D_QK = 256
_D_V = 128
_H_QK = 32
_H_V = 16

_BL = 2000
_BE = 2000


def _phase1_body(k_ref, v_ref, kv_ref):
    @pl.when(pl.program_id(0) == 0)
    def _init():
        kv_ref[...] = jnp.zeros_like(kv_ref)

    kb = k_ref[...].astype(jnp.bfloat16)
    vb = v_ref[...].astype(jnp.bfloat16)
    kv_ref[...] += jax.lax.dot_general(
        kb, vb, (((0,), (0,)), ((), ())), preferred_element_type=jnp.float32)


def _phase2_body(q_ref, kv_ref, rep_ref, out_ref):
    qb = q_ref[...].astype(jnp.bfloat16)
    row_h = jax.lax.broadcasted_iota(jnp.int32, (_D_QK, _D_V), 0) // _H_QK
    col_h = jax.lax.broadcasted_iota(jnp.int32, (_D_QK, _D_V), 1) // _H_V
    mask = row_h == col_h
    kv_bd = jnp.where(mask, kv_ref[...], 0.0).astype(jnp.bfloat16)
    rep_b = rep_ref[...].astype(jnp.bfloat16)
    num = jax.lax.dot_general(
        qb, kv_bd, (((1,), (0,)), ((), ())), preferred_element_type=jnp.float32)
    den = jax.lax.dot_general(
        qb, rep_b, (((1,), (0,)), ((), ())), preferred_element_type=jnp.float32)
    den = jnp.where(den == 0.0, 1e-5, den)
    out_ref[...] = num / den


def kernel(query, key_t, value):
    b, n, _ = query.shape
    q = query.reshape(n, _D_QK)
    k = key_t.reshape(n, _D_QK)
    v = value.reshape(n, _D_V)

    ks_ = jnp.stack(jnp.split(key_t, 8, axis=-1), axis=0).sum(axis=2)
    ks_ = jax.lax.optimization_barrier(ks_)
    ks_col = ks_.reshape(_D_QK, 1)
    rep = jnp.where(
        (jnp.arange(_D_QK)[:, None] // _H_QK) == (jnp.arange(_D_V)[None, :] // _H_V),
        ks_col, jnp.float32(0.0))

    nb = n // _BL
    kv = pl.pallas_call(
        _phase1_body,
        grid=(nb,),
        in_specs=[
            pl.BlockSpec((_BL, _D_QK), lambda i: (i, 0)),
            pl.BlockSpec((_BL, _D_V), lambda i: (i, 0)),
        ],
        out_specs=pl.BlockSpec((_D_QK, _D_V), lambda i: (0, 0)),
        out_shape=jax.ShapeDtypeStruct((_D_QK, _D_V), jnp.float32),
    )(k, v)

    ne = n // _BE
    att = pl.pallas_call(
        _phase2_body,
        grid=(ne,),
        in_specs=[
            pl.BlockSpec((_BE, _D_QK), lambda i: (i, 0)),
            pl.BlockSpec((_D_QK, _D_V), lambda i: (0, 0)),
            pl.BlockSpec((_D_QK, _D_V), lambda i: (0, 0)),
        ],
        out_specs=pl.BlockSpec((_BE, _D_V), lambda i: (i, 0)),
        out_shape=jax.ShapeDtypeStruct((n, _D_V), jnp.float32),
    )(q, kv, rep)

    return att.reshape(b, n, _D_V)


# speedup vs baseline: 2.2092x; 1.0400x over previous
import jax
import jax.numpy as jnp
from jax.experimental import pallas as pl

_D_QK = 256
_D_V = 128
_H_QK = 32
_H_V = 16

_BL = 10000
_BE = 10000


def _phase1_body(k_ref, v_ref, kv_ref):
    @pl.when(pl.program_id(0) == 0)
    def _init():
        kv_ref[...] = jnp.zeros_like(kv_ref)

    kb = k_ref[...].astype(jnp.bfloat16)
    vb = v_ref[...].astype(jnp.bfloat16)
    kv_ref[...] += jax.lax.dot_general(
        kb, vb, (((0,), (0,)), ((), ())), preferred_element_type=jnp.float32)


def _phase2_body(q_ref, kv_ref, rep_ref, out_ref):
    qb = q_ref[...].astype(jnp.bfloat16)
    row_h = jax.lax.broadcasted_iota(jnp.int32, (_D_QK, _D_V), 0) // _H_QK
    col_h = jax.lax.broadcasted_iota(jnp.int32, (_D_QK, _D_V), 1) // _H_V
    mask = row_h == col_h
    kv_bd = jnp.where(mask, kv_ref[...], 0.0).astype(jnp.bfloat16)
    rep_b = rep_ref[...].astype(jnp.bfloat16)
    num = jax.lax.dot_general(
        qb, kv_bd, (((1,), (0,)), ((), ())), preferred_element_type=jnp.float32)
    den = jax.lax.dot_general(
        qb, rep_b, (((1,), (0,)), ((), ())), preferred_element_type=jnp.float32)
    den = jnp.where(den == 0.0, 1e-5, den)
    out_ref[...] = num / den


def kernel(query, key_t, value):
    b, n, _ = query.shape
    q = query.reshape(n, _D_QK)
    k = key_t.reshape(n, _D_QK)
    v = value.reshape(n, _D_V)

    ks_ = jnp.stack(jnp.split(key_t, 8, axis=-1), axis=0).sum(axis=2)
    ks_ = jax.lax.optimization_barrier(ks_)
    ks_col = ks_.reshape(_D_QK, 1)
    rep = jnp.where(
        (jnp.arange(_D_QK)[:, None] // _H_QK) == (jnp.arange(_D_V)[None, :] // _H_V),
        ks_col, jnp.float32(0.0))

    nb = n // _BL
    kv = pl.pallas_call(
        _phase1_body,
        grid=(nb,),
        in_specs=[
            pl.BlockSpec((_BL, _D_QK), lambda i: (i, 0)),
            pl.BlockSpec((_BL, _D_V), lambda i: (i, 0)),
        ],
        out_specs=pl.BlockSpec((_D_QK, _D_V), lambda i: (0, 0)),
        out_shape=jax.ShapeDtypeStruct((_D_QK, _D_V), jnp.float32),
    )(k, v)

    ne = n // _BE
    att = pl.pallas_call(
        _phase2_body,
        grid=(ne,),
        in_specs=[
            pl.BlockSpec((_BE, _D_QK), lambda i: (i, 0)),
            pl.BlockSpec((_D_QK, _D_V), lambda i: (0, 0)),
            pl.BlockSpec((_D_QK, _D_V), lambda i: (0, 0)),
        ],
        out_specs=pl.BlockSpec((_BE, _D_V), lambda i: (i, 0)),
        out_shape=jax.ShapeDtypeStruct((n, _D_V), jnp.float32),
    )(q, kv, rep)

    return att.reshape(b, n, _D_V)
